# baseline (speedup 1.0000x reference)
import functools

import jax
import jax.numpy as jnp
from jax import lax
from jax.experimental import pallas as pl
from jax.experimental.pallas import tpu as pltpu
from jax.experimental.pallas import tpu_sc as plsc

DEPTH = 64
SEQ_LEN = 200
LANES = 16
NC = 2
NS = 16
NW = NC * NS

ROWS_PER_STREAM = 128
STREAMS_PER_MACRO = 8
MACRO = ROWS_PER_STREAM * STREAMS_PER_MACRO


@functools.cache
def _build(n_rows):
    assert n_rows % (NW * MACRO) == 0
    b_per_w = n_rows // NW
    n_macro = b_per_w // MACRO
    mesh = plsc.VectorSubcoreMesh(core_axis_name="c", subcore_axis_name="s")

    @functools.partial(
        pl.kernel,
        mesh=mesh,
        out_type=jax.ShapeDtypeStruct((n_rows, DEPTH), jnp.float32),
        scratch_types=[
            pltpu.VMEM((STREAMS_PER_MACRO, ROWS_PER_STREAM), jnp.int32),
            pltpu.VMEM((MACRO, DEPTH), jnp.float32),
            pltpu.VMEM((SEQ_LEN, DEPTH), jnp.float32),
            pltpu.SemaphoreType.DMA,
        ],
        compiler_params=pltpu.CompilerParams(use_tc_tiling_on_sc=False),
    )
    def run(seq_hbm, tok_hbm, pos_hbm, out_hbm, idx_v, rows_v, pos_v, sem):
        wid = lax.axis_index("s") * NC + lax.axis_index("c")
        base_w = wid * b_per_w
        pltpu.sync_copy(pos_hbm, pos_v)

        def macro_body(m, p0):
            base = base_w + m * MACRO
            row0 = pl.multiple_of(base // ROWS_PER_STREAM, 8)
            pltpu.sync_copy(
                seq_hbm.at[pl.ds(row0, STREAMS_PER_MACRO)],
                idx_v,
            )
            copies = [
                pltpu.async_copy(
                    tok_hbm.at[idx_v.at[k]],
                    rows_v.at[pl.ds(k * ROWS_PER_STREAM, ROWS_PER_STREAM)],
                    sem,
                )
                for k in range(STREAMS_PER_MACRO)
            ]
            for c in copies:
                c.wait()

            def row_body(r, p):
                for col in range(DEPTH // LANES):
                    sl = pl.ds(col * LANES, LANES)
                    rows_v[r, sl] = rows_v[r, sl] + pos_v[p, sl]
                pn = p + 1
                return jnp.where(pn == SEQ_LEN, 0, pn)

            p_end = lax.fori_loop(0, MACRO, row_body, p0)
            pltpu.sync_copy(rows_v, out_hbm.at[pl.ds(base, MACRO)])
            return p_end

        lax.fori_loop(0, n_macro, macro_body, jnp.int32(0))

    return run


def kernel(seq, token_table, pos_table):
    b, l = seq.shape
    n = b * l
    seq2 = seq.reshape(n // ROWS_PER_STREAM, ROWS_PER_STREAM)
    out = _build(n)(seq2, token_table, pos_table)
    return out.reshape(b, l, DEPTH)
